# Initial kernel scaffold; baseline (speedup 1.0000x reference)
import jax
import jax.numpy as jnp
from jax.experimental import pallas as pl


def kernel(conv_w, conv_b, fc_w, fc_b, paired_img):
    raise NotImplementedError("write your pallas kernel here")


# speedup vs baseline: 2.5602x; 2.5602x over previous
import jax
import jax.numpy as jnp
from jax.experimental import pallas as pl
from jax.experimental.pallas import tpu as pltpu

_BBLK = 64
_GW = 16


def _fused_body(x_ref, wp_ref, wh_ref, b_ref, o_ref, *, Bblk, H, W, Cin, Cmid):
    ng = W // _GW
    pw = (_GW + 2) * Cin
    xb = x_ref[...]
    zrow = jnp.zeros((Bblk, 1, (W + 2) * Cin), xb.dtype)
    xm = jnp.concatenate([zrow, xb[:, : H - 1]], axis=1)
    xp = jnp.concatenate([xb[:, 1:], zrow], axis=1)

    grps = []
    for g in range(ng):
        lo = g * _GW * Cin
        grps.append(jnp.concatenate(
            [xm[:, :, lo:lo + pw], xb[:, :, lo:lo + pw], xp[:, :, lo:lo + pw]],
            axis=-1))
    lhs = jnp.concatenate(grps, axis=0).reshape(ng * Bblk * H, 3 * pw)

    y = jnp.dot(lhs, wp_ref[...], preferred_element_type=jnp.float32)
    cb = b_ref[0:1, :]
    act = jnp.maximum(y + cb, 0.0)
    s = act.reshape(ng, Bblk, H, _GW * Cmid).sum(axis=(0, 2))
    fb = b_ref[1:2, 0:128]
    o_ref[...] = jnp.dot(s, wh_ref[...],
                         preferred_element_type=jnp.float32) + fb


def _build_consts(conv_w, conv_b, fc_w, fc_b, H, W):
    KH, KW, Cin, Cmid = conv_w.shape
    n_cls = fc_w.shape[-1]
    pw = (_GW + 2) * Cin
    wp_idx = jnp.arange(_GW + 2)
    wo_idx = jnp.arange(_GW)
    dw_idx = jnp.arange(KW)
    sel = (wp_idx[:, None, None] == wo_idx[None, :, None] + dw_idx[None, None, :])
    wpatch = jnp.einsum("pvd,hdic->hpivc", sel.astype(jnp.float32),
                        conv_w.astype(jnp.float32))
    wpatch = wpatch.reshape(KH * pw, _GW * Cmid).astype(jnp.bfloat16)

    ssum = jnp.tile(jnp.eye(Cmid, dtype=jnp.float32), (_GW, 1))
    whead = (ssum @ fc_w[:Cmid].astype(jnp.float32)) * (1.0 / (H * W))
    whead = jnp.pad(whead, ((0, 0), (0, 128 - n_cls)))

    bias2 = jnp.zeros((8, _GW * Cmid), jnp.float32)
    bias2 = bias2.at[0, :].set(jnp.tile(conv_b.astype(jnp.float32), _GW))
    bias2 = bias2.at[1, :n_cls].set(fc_b.astype(jnp.float32))
    return wpatch, whead, bias2


def kernel(conv_w, conv_b, fc_w, fc_b, paired_img):
    B, Cin, H, W = paired_img.shape
    KH, KW, _, Cmid = conv_w.shape
    n_cls = fc_w.shape[-1]
    Bblk = min(_BBLK, B)
    nB = pl.cdiv(B, Bblk)
    B_pad = nB * Bblk

    wpatch, whead, bias2 = _build_consts(conv_w, conv_b, fc_w, fc_b, H, W)

    x = jnp.transpose(paired_img, (0, 2, 3, 1))
    x = jnp.pad(x, ((0, B_pad - B), (0, 0), (1, 1), (0, 0)))
    x = x.reshape(B_pad, H, (W + 2) * Cin).astype(jnp.bfloat16)

    body = lambda *refs: _fused_body(*refs, Bblk=Bblk, H=H, W=W,
                                     Cin=Cin, Cmid=Cmid)
    out = pl.pallas_call(
        body,
        out_shape=jax.ShapeDtypeStruct((B_pad, 128), jnp.float32),
        grid_spec=pltpu.PrefetchScalarGridSpec(
            num_scalar_prefetch=0,
            grid=(nB,),
            in_specs=[
                pl.BlockSpec((Bblk, H, (W + 2) * Cin), lambda b: (b, 0, 0)),
                pl.BlockSpec(wpatch.shape, lambda b: (0, 0)),
                pl.BlockSpec(whead.shape, lambda b: (0, 0)),
                pl.BlockSpec(bias2.shape, lambda b: (0, 0)),
            ],
            out_specs=pl.BlockSpec((Bblk, 128), lambda b: (b, 0)),
        ),
        compiler_params=pltpu.CompilerParams(
            dimension_semantics=("parallel",)),
    )(x, wpatch, whead, bias2)
    return out[:B, :n_cls]


# speedup vs baseline: 2.5859x; 1.0100x over previous
import jax
import jax.numpy as jnp
from jax.experimental import pallas as pl
from jax.experimental.pallas import tpu as pltpu

_BBLK = 128
_GW = 16


def _fused_body(x_ref, wp_ref, wh_ref, b_ref, o_ref, *, Bblk, H, W, Cin, Cmid):
    ng = W // _GW
    pw = (_GW + 2) * Cin
    xb = x_ref[...]
    zrow = jnp.zeros((Bblk, 1, (W + 2) * Cin), xb.dtype)
    xm = jnp.concatenate([zrow, xb[:, : H - 1]], axis=1)
    xp = jnp.concatenate([xb[:, 1:], zrow], axis=1)

    grps = []
    for g in range(ng):
        lo = g * _GW * Cin
        grps.append(jnp.concatenate(
            [xm[:, :, lo:lo + pw], xb[:, :, lo:lo + pw], xp[:, :, lo:lo + pw]],
            axis=-1))
    lhs = jnp.concatenate(grps, axis=0).reshape(ng * Bblk * H, 3 * pw)

    y = jnp.dot(lhs, wp_ref[...], preferred_element_type=jnp.float32)
    cb = b_ref[0:1, :]
    act = jnp.maximum(y + cb, 0.0)
    s = act.reshape(ng, Bblk, H, _GW * Cmid).sum(axis=(0, 2))
    fb = b_ref[1:2, 0:128]
    o_ref[...] = jnp.dot(s, wh_ref[...],
                         preferred_element_type=jnp.float32) + fb


def _build_consts(conv_w, conv_b, fc_w, fc_b, H, W):
    KH, KW, Cin, Cmid = conv_w.shape
    n_cls = fc_w.shape[-1]
    pw = (_GW + 2) * Cin
    wp_idx = jnp.arange(_GW + 2)
    wo_idx = jnp.arange(_GW)
    dw_idx = jnp.arange(KW)
    sel = (wp_idx[:, None, None] == wo_idx[None, :, None] + dw_idx[None, None, :])
    wpatch = jnp.einsum("pvd,hdic->hpivc", sel.astype(jnp.float32),
                        conv_w.astype(jnp.float32))
    wpatch = wpatch.reshape(KH * pw, _GW * Cmid).astype(jnp.bfloat16)

    ssum = jnp.tile(jnp.eye(Cmid, dtype=jnp.float32), (_GW, 1))
    whead = (ssum @ fc_w[:Cmid].astype(jnp.float32)) * (1.0 / (H * W))
    whead = jnp.pad(whead, ((0, 0), (0, 128 - n_cls)))

    bias2 = jnp.zeros((8, _GW * Cmid), jnp.float32)
    bias2 = bias2.at[0, :].set(jnp.tile(conv_b.astype(jnp.float32), _GW))
    bias2 = bias2.at[1, :n_cls].set(fc_b.astype(jnp.float32))
    return wpatch, whead, bias2


def kernel(conv_w, conv_b, fc_w, fc_b, paired_img):
    B, Cin, H, W = paired_img.shape
    KH, KW, _, Cmid = conv_w.shape
    n_cls = fc_w.shape[-1]
    Bblk = min(_BBLK, B)
    nB = pl.cdiv(B, Bblk)
    B_pad = nB * Bblk

    wpatch, whead, bias2 = _build_consts(conv_w, conv_b, fc_w, fc_b, H, W)

    x = jnp.transpose(paired_img, (0, 2, 3, 1))
    x = jnp.pad(x, ((0, B_pad - B), (0, 0), (1, 1), (0, 0)))
    x = x.reshape(B_pad, H, (W + 2) * Cin).astype(jnp.bfloat16)

    body = lambda *refs: _fused_body(*refs, Bblk=Bblk, H=H, W=W,
                                     Cin=Cin, Cmid=Cmid)
    out = pl.pallas_call(
        body,
        out_shape=jax.ShapeDtypeStruct((B_pad, 128), jnp.float32),
        grid_spec=pltpu.PrefetchScalarGridSpec(
            num_scalar_prefetch=0,
            grid=(nB,),
            in_specs=[
                pl.BlockSpec((Bblk, H, (W + 2) * Cin), lambda b: (b, 0, 0)),
                pl.BlockSpec(wpatch.shape, lambda b: (0, 0)),
                pl.BlockSpec(whead.shape, lambda b: (0, 0)),
                pl.BlockSpec(bias2.shape, lambda b: (0, 0)),
            ],
            out_specs=pl.BlockSpec((Bblk, 128), lambda b: (b, 0)),
        ),
        compiler_params=pltpu.CompilerParams(
            dimension_semantics=("parallel",)),
    )(x, wpatch, whead, bias2)
    return out[:B, :n_cls]


# speedup vs baseline: 3.2066x; 1.2400x over previous
import jax
import jax.numpy as jnp
from jax.experimental import pallas as pl
from jax.experimental.pallas import tpu as pltpu

_BBLK = 128
_GW = 16


def _fused_body(x_ref, w0_ref, w1_ref, wh_ref, b_ref, o_ref, *,
                Bblk, H, W, Cin, Cmid):
    pw = (_GW + 1) * Cin
    xb = x_ref[...]
    zrow = jnp.zeros((Bblk, 1, W * Cin), xb.dtype)
    xm = jnp.concatenate([zrow, xb[:, : H - 1]], axis=1)
    xp = jnp.concatenate([xb[:, 1:], zrow], axis=1)

    lo1 = W * Cin - pw
    p0 = jnp.concatenate(
        [xm[:, :, :pw], xb[:, :, :pw], xp[:, :, :pw]], axis=-1)
    p1 = jnp.concatenate(
        [xm[:, :, lo1:], xb[:, :, lo1:], xp[:, :, lo1:]], axis=-1)
    p0 = p0.reshape(Bblk * H, 3 * pw)
    p1 = p1.reshape(Bblk * H, 3 * pw)

    y0 = jnp.dot(p0, w0_ref[...], preferred_element_type=jnp.float32)
    y1 = jnp.dot(p1, w1_ref[...], preferred_element_type=jnp.float32)
    cb = b_ref[0:1, :]
    act = (jnp.maximum(y0 + cb, 0.0) + jnp.maximum(y1 + cb, 0.0))
    s = act.reshape(Bblk, H, _GW * Cmid).sum(axis=1)
    fb = b_ref[1:2, 0:128]
    o_ref[...] = jnp.dot(s, wh_ref[...],
                         preferred_element_type=jnp.float32) + fb


def _build_consts(conv_w, conv_b, fc_w, fc_b, H, W):
    KH, KW, Cin, Cmid = conv_w.shape
    n_cls = fc_w.shape[-1]
    pw = (_GW + 1) * Cin
    conv_w = conv_w.astype(jnp.float32)

    wp_idx = jnp.arange(_GW + 1)
    wo_idx = jnp.arange(_GW)
    dw_idx = jnp.arange(KW)
    sel0 = (wp_idx[:, None, None] == wo_idx[None, :, None] + dw_idx[None, None, :] - 1)
    sel1 = (wp_idx[:, None, None] == wo_idx[None, :, None] + dw_idx[None, None, :])
    w0 = jnp.einsum("pvd,hdic->hpivc", sel0.astype(jnp.float32), conv_w)
    w1 = jnp.einsum("pvd,hdic->hpivc", sel1.astype(jnp.float32), conv_w)
    w0 = w0.reshape(KH * pw, _GW * Cmid).astype(jnp.bfloat16)
    w1 = w1.reshape(KH * pw, _GW * Cmid).astype(jnp.bfloat16)

    ssum = jnp.tile(jnp.eye(Cmid, dtype=jnp.float32), (_GW, 1))
    whead = (ssum @ fc_w[:Cmid].astype(jnp.float32)) * (1.0 / (H * W))
    whead = jnp.pad(whead, ((0, 0), (0, 128 - n_cls)))

    bias2 = jnp.zeros((8, _GW * Cmid), jnp.float32)
    bias2 = bias2.at[0, :].set(jnp.tile(conv_b.astype(jnp.float32), _GW))
    bias2 = bias2.at[1, :n_cls].set(fc_b.astype(jnp.float32))
    return w0, w1, whead, bias2


def kernel(conv_w, conv_b, fc_w, fc_b, paired_img):
    B, Cin, H, W = paired_img.shape
    KH, KW, _, Cmid = conv_w.shape
    n_cls = fc_w.shape[-1]
    Bblk = min(_BBLK, B)
    nB = pl.cdiv(B, Bblk)
    B_pad = nB * Bblk

    w0, w1, whead, bias2 = _build_consts(conv_w, conv_b, fc_w, fc_b, H, W)

    x = jnp.transpose(paired_img, (0, 2, 3, 1))
    x = x.reshape(B, H, W * Cin).astype(jnp.bfloat16)
    if B_pad != B:
        x = jnp.pad(x, ((0, B_pad - B), (0, 0), (0, 0)))

    body = lambda *refs: _fused_body(*refs, Bblk=Bblk, H=H, W=W,
                                     Cin=Cin, Cmid=Cmid)
    out = pl.pallas_call(
        body,
        out_shape=jax.ShapeDtypeStruct((B_pad, 128), jnp.float32),
        grid_spec=pltpu.PrefetchScalarGridSpec(
            num_scalar_prefetch=0,
            grid=(nB,),
            in_specs=[
                pl.BlockSpec((Bblk, H, W * Cin), lambda b: (b, 0, 0)),
                pl.BlockSpec(w0.shape, lambda b: (0, 0)),
                pl.BlockSpec(w1.shape, lambda b: (0, 0)),
                pl.BlockSpec(whead.shape, lambda b: (0, 0)),
                pl.BlockSpec(bias2.shape, lambda b: (0, 0)),
            ],
            out_specs=pl.BlockSpec((Bblk, 128), lambda b: (b, 0)),
        ),
        compiler_params=pltpu.CompilerParams(
            dimension_semantics=("parallel",)),
    )(x, w0, w1, whead, bias2)
    return out[:B, :n_cls]
